# baseline (device time: 423212 ns/iter reference)
import jax
import jax.numpy as jnp
from jax import lax
from jax.experimental import pallas as pl
from jax.experimental.pallas import tpu as pltpu

N_DEV = 32
M = 4096
M_PER = M // N_DEV
N_COLS = 2048
N_HALF = N_COLS // 2
N_STEPS = N_DEV - 1
LOG2_DEV = 5


def kernel(x, w_mat):
    def body(x_ref, w_ref, out_ref,
             send_r, recv_r, send_l, recv_l,
             send_sems_r, recv_sems_r, send_sems_l, recv_sems_l,
             credit_r, credit_l,
             amax_send, amax_recv, amax_send_sems, amax_recv_sems):
        d = lax.axis_index("i")
        left = jnp.mod(d - 1, N_DEV)
        right = jnp.mod(d + 1, N_DEV)

        barrier_sem = pltpu.get_barrier_semaphore()
        for nbr in (left, right):
            pl.semaphore_signal(barrier_sem, inc=1, device_id=(nbr,),
                                device_id_type=pl.DeviceIdType.MESH)
        pl.semaphore_wait(barrier_sem, 2)

        def partial(c, lo):
            return jnp.dot(
                x_ref[pl.ds(c * M_PER, M_PER), :],
                w_ref[:, lo:lo + N_HALF],
                preferred_element_type=jnp.float32,
            )

        send_r[0, :, :] = partial(jnp.mod(d - 1, N_DEV), 0)
        send_l[0, :, :] = partial(jnp.mod(d + 1, N_DEV), N_HALF)

        y_r = None
        y_l = None
        for s in range(N_STEPS):
            sp = s % 2
            if s >= 2:
                pl.semaphore_wait(credit_r, 1)
                pl.semaphore_wait(credit_l, 1)
            rdma_r = pltpu.make_async_remote_copy(
                src_ref=send_r.at[sp], dst_ref=recv_r.at[sp],
                send_sem=send_sems_r.at[sp], recv_sem=recv_sems_r.at[sp],
                device_id=(right,), device_id_type=pl.DeviceIdType.MESH)
            rdma_l = pltpu.make_async_remote_copy(
                src_ref=send_l.at[sp], dst_ref=recv_l.at[sp],
                send_sem=send_sems_l.at[sp], recv_sem=recv_sems_l.at[sp],
                device_id=(left,), device_id_type=pl.DeviceIdType.MESH)
            rdma_r.start()
            rdma_l.start()

            p_r = partial(jnp.mod(d - 2 - s, N_DEV), 0)
            p_l = partial(jnp.mod(d + 2 + s, N_DEV), N_HALF)

            rdma_r.wait()
            rdma_l.wait()

            if s < N_STEPS - 1:
                nsp = (s + 1) % 2
                send_r[nsp, :, :] = recv_r[sp, :, :] + p_r
                send_l[nsp, :, :] = recv_l[sp, :, :] + p_l
                if s <= N_STEPS - 3:
                    pl.semaphore_signal(credit_r, inc=1, device_id=(left,),
                                        device_id_type=pl.DeviceIdType.MESH)
                    pl.semaphore_signal(credit_l, inc=1, device_id=(right,),
                                        device_id_type=pl.DeviceIdType.MESH)
            else:
                y_r = jnp.maximum(recv_r[sp, :, :] + p_r, 0.0)
                y_l = jnp.maximum(recv_l[sp, :, :] + p_l, 0.0)

        out_ref[:, :N_HALF] = y_r
        out_ref[:, N_HALF:] = y_l

        amax = jnp.maximum(jnp.max(y_r), jnp.max(y_l))
        for r in range(LOG2_DEV):
            partner = jnp.bitwise_xor(d, 1 << r)
            amax_send[r, :, :] = jnp.full((8, 128), amax, jnp.float32)
            ex = pltpu.make_async_remote_copy(
                src_ref=amax_send.at[r], dst_ref=amax_recv.at[r],
                send_sem=amax_send_sems.at[r], recv_sem=amax_recv_sems.at[r],
                device_id=(partner,), device_id_type=pl.DeviceIdType.MESH)
            ex.start()
            ex.wait()
            amax = jnp.maximum(amax, jnp.max(amax_recv[r, :, :]))

        scale = amax / 127.0
        q = jnp.clip(jnp.round(out_ref[:, :] / scale), 0.0, 127.0)
        out_ref[:, :] = q * scale

    return pl.pallas_call(
        body,
        out_shape=jax.ShapeDtypeStruct((M_PER, N_COLS), jnp.float32),
        in_specs=[
            pl.BlockSpec(memory_space=pltpu.VMEM),
            pl.BlockSpec(memory_space=pltpu.VMEM),
        ],
        out_specs=pl.BlockSpec(memory_space=pltpu.VMEM),
        scratch_shapes=[
            pltpu.VMEM((2, M_PER, N_HALF), jnp.float32),
            pltpu.VMEM((2, M_PER, N_HALF), jnp.float32),
            pltpu.VMEM((2, M_PER, N_HALF), jnp.float32),
            pltpu.VMEM((2, M_PER, N_HALF), jnp.float32),
            pltpu.SemaphoreType.DMA((2,)),
            pltpu.SemaphoreType.DMA((2,)),
            pltpu.SemaphoreType.DMA((2,)),
            pltpu.SemaphoreType.DMA((2,)),
            pltpu.SemaphoreType.REGULAR,
            pltpu.SemaphoreType.REGULAR,
            pltpu.VMEM((LOG2_DEV, 8, 128), jnp.float32),
            pltpu.VMEM((LOG2_DEV, 8, 128), jnp.float32),
            pltpu.SemaphoreType.DMA((LOG2_DEV,)),
            pltpu.SemaphoreType.DMA((LOG2_DEV,)),
        ],
        compiler_params=pltpu.CompilerParams(collective_id=0),
    )(x, w_mat)


# device time: 252170 ns/iter; 1.6783x vs baseline; 1.6783x over previous
import jax
import jax.numpy as jnp
from jax import lax
from jax.experimental import pallas as pl
from jax.experimental.pallas import tpu as pltpu

N_DEV = 32
M = 4096
M_PER = M // N_DEV
N_COLS = 2048
N_HALF = N_COLS // 2
N_STEPS = N_DEV - 1
LOG2_DEV = 5


def _ring_tables():
    logical = []
    for z in range(4):
        for y in range(4):
            for x in ((0, 1) if y % 2 == 0 else (1, 0)):
                logical.append((x, y, z))
    bous = [(0, 0), (1, 0), (2, 0), (3, 0), (3, 1), (2, 1), (1, 1), (0, 1),
            (0, 2), (1, 2), (2, 2), (3, 2), (3, 3), (2, 3), (1, 3), (0, 3)]
    cycle = [(0, y, z) for (y, z) in bous] + \
            [(1, y, z) for (y, z) in reversed(bous)]
    sigma = [logical.index(c) for c in cycle]
    inv = [0] * N_DEV
    for r, l in enumerate(sigma):
        inv[l] = r
    return sigma, inv


_SIGMA, _INV = _ring_tables()


def kernel(x, w_mat):
    def body(sigma_ref, inv_ref, x_ref, w_ref, out_ref,
             send_r, recv_r, send_l, recv_l,
             send_sems_r, recv_sems_r, send_sems_l, recv_sems_l,
             credit_r, credit_l,
             amax_send, amax_recv, amax_send_sems, amax_recv_sems):
        d = lax.axis_index("i")
        rho = inv_ref[d]
        right = sigma_ref[jnp.mod(rho + 1, N_DEV)]
        left = sigma_ref[jnp.mod(rho - 1, N_DEV)]

        barrier_sem = pltpu.get_barrier_semaphore()
        for nbr in (left, right):
            pl.semaphore_signal(barrier_sem, inc=1, device_id=(nbr,),
                                device_id_type=pl.DeviceIdType.MESH)
        pl.semaphore_wait(barrier_sem, 2)

        def partial(c, lo):
            return jnp.dot(
                x_ref[pl.ds(c * M_PER, M_PER), :],
                w_ref[:, lo:lo + N_HALF],
                preferred_element_type=jnp.float32,
            )

        send_r[0, :, :] = partial(sigma_ref[jnp.mod(rho - 1, N_DEV)], 0)
        send_l[0, :, :] = partial(sigma_ref[jnp.mod(rho + 1, N_DEV)], N_HALF)

        y_r = None
        y_l = None
        for s in range(N_STEPS):
            sp = s % 2
            if s >= 2:
                pl.semaphore_wait(credit_r, 1)
                pl.semaphore_wait(credit_l, 1)
            rdma_r = pltpu.make_async_remote_copy(
                src_ref=send_r.at[sp], dst_ref=recv_r.at[sp],
                send_sem=send_sems_r.at[sp], recv_sem=recv_sems_r.at[sp],
                device_id=(right,), device_id_type=pl.DeviceIdType.MESH)
            rdma_l = pltpu.make_async_remote_copy(
                src_ref=send_l.at[sp], dst_ref=recv_l.at[sp],
                send_sem=send_sems_l.at[sp], recv_sem=recv_sems_l.at[sp],
                device_id=(left,), device_id_type=pl.DeviceIdType.MESH)
            rdma_r.start()
            rdma_l.start()

            p_r = partial(sigma_ref[jnp.mod(rho - 2 - s, N_DEV)], 0)
            p_l = partial(sigma_ref[jnp.mod(rho + 2 + s, N_DEV)], N_HALF)

            rdma_r.wait()
            rdma_l.wait()

            if s < N_STEPS - 1:
                nsp = (s + 1) % 2
                send_r[nsp, :, :] = recv_r[sp, :, :] + p_r
                send_l[nsp, :, :] = recv_l[sp, :, :] + p_l
                if s <= N_STEPS - 3:
                    pl.semaphore_signal(credit_r, inc=1, device_id=(left,),
                                        device_id_type=pl.DeviceIdType.MESH)
                    pl.semaphore_signal(credit_l, inc=1, device_id=(right,),
                                        device_id_type=pl.DeviceIdType.MESH)
            else:
                y_r = jnp.maximum(recv_r[sp, :, :] + p_r, 0.0)
                y_l = jnp.maximum(recv_l[sp, :, :] + p_l, 0.0)

        out_ref[:, :N_HALF] = y_r
        out_ref[:, N_HALF:] = y_l

        amax = jnp.maximum(jnp.max(y_r), jnp.max(y_l))
        for r in range(LOG2_DEV):
            partner = jnp.bitwise_xor(d, 1 << r)
            amax_send[r, :, :] = jnp.full((8, 128), amax, jnp.float32)
            ex = pltpu.make_async_remote_copy(
                src_ref=amax_send.at[r], dst_ref=amax_recv.at[r],
                send_sem=amax_send_sems.at[r], recv_sem=amax_recv_sems.at[r],
                device_id=(partner,), device_id_type=pl.DeviceIdType.MESH)
            ex.start()
            ex.wait()
            amax = jnp.maximum(amax, jnp.max(amax_recv[r, :, :]))

        scale = amax / 127.0
        q = jnp.clip(jnp.round(out_ref[:, :] / scale), 0.0, 127.0)
        out_ref[:, :] = q * scale

    sigma_arr = jnp.array(_SIGMA, dtype=jnp.int32)
    inv_arr = jnp.array(_INV, dtype=jnp.int32)

    return pl.pallas_call(
        body,
        out_shape=jax.ShapeDtypeStruct((M_PER, N_COLS), jnp.float32),
        in_specs=[
            pl.BlockSpec(memory_space=pltpu.SMEM),
            pl.BlockSpec(memory_space=pltpu.SMEM),
            pl.BlockSpec(memory_space=pltpu.VMEM),
            pl.BlockSpec(memory_space=pltpu.VMEM),
        ],
        out_specs=pl.BlockSpec(memory_space=pltpu.VMEM),
        scratch_shapes=[
            pltpu.VMEM((2, M_PER, N_HALF), jnp.float32),
            pltpu.VMEM((2, M_PER, N_HALF), jnp.float32),
            pltpu.VMEM((2, M_PER, N_HALF), jnp.float32),
            pltpu.VMEM((2, M_PER, N_HALF), jnp.float32),
            pltpu.SemaphoreType.DMA((2,)),
            pltpu.SemaphoreType.DMA((2,)),
            pltpu.SemaphoreType.DMA((2,)),
            pltpu.SemaphoreType.DMA((2,)),
            pltpu.SemaphoreType.REGULAR,
            pltpu.SemaphoreType.REGULAR,
            pltpu.VMEM((LOG2_DEV, 8, 128), jnp.float32),
            pltpu.VMEM((LOG2_DEV, 8, 128), jnp.float32),
            pltpu.SemaphoreType.DMA((LOG2_DEV,)),
            pltpu.SemaphoreType.DMA((LOG2_DEV,)),
        ],
        compiler_params=pltpu.CompilerParams(collective_id=0),
    )(sigma_arr, inv_arr, x, w_mat)


# device time: 199221 ns/iter; 2.1243x vs baseline; 1.2658x over previous
import jax
import jax.numpy as jnp
from jax import lax
from jax.experimental import pallas as pl
from jax.experimental.pallas import tpu as pltpu

N_DEV = 32
M = 4096
M_PER = M // N_DEV
N_COLS = 2048
N_HALF = N_COLS // 2
N_SUB = N_HALF // 2
N_STEPS = N_DEV - 1
LOG2_DEV = 5


def _ring_tables():
    logical = []
    for z in range(4):
        for y in range(4):
            for x in ((0, 1) if y % 2 == 0 else (1, 0)):
                logical.append((x, y, z))
    bous = [(0, 0), (1, 0), (2, 0), (3, 0), (3, 1), (2, 1), (1, 1), (0, 1),
            (0, 2), (1, 2), (2, 2), (3, 2), (3, 3), (2, 3), (1, 3), (0, 3)]
    cycle = [(0, y, z) for (y, z) in bous] + \
            [(1, y, z) for (y, z) in reversed(bous)]
    sigma = [logical.index(c) for c in cycle]
    inv = [0] * N_DEV
    for r, l in enumerate(sigma):
        inv[l] = r
    return sigma, inv


_SIGMA, _INV = _ring_tables()


def kernel(x, w_mat):
    def body(sigma_ref, inv_ref, x_ref, w_ref, out_ref,
             send_r, recv_r, send_l, recv_l,
             send_sems_r, recv_sems_r, send_sems_l, recv_sems_l,
             creditA_r, creditB_r, creditA_l, creditB_l,
             amax_send, amax_recv, amax_send_sems, amax_recv_sems):
        d = lax.axis_index("i")
        rho = inv_ref[d]
        right = sigma_ref[jnp.mod(rho + 1, N_DEV)]
        left = sigma_ref[jnp.mod(rho - 1, N_DEV)]

        barrier_sem = pltpu.get_barrier_semaphore()
        for nbr in (left, right):
            pl.semaphore_signal(barrier_sem, inc=1, device_id=(nbr,),
                                device_id_type=pl.DeviceIdType.MESH)
        pl.semaphore_wait(barrier_sem, 2)

        def partial(c, lo):
            return jnp.dot(
                x_ref[pl.ds(c * M_PER, M_PER), :],
                w_ref[:, lo:lo + N_SUB],
                preferred_element_type=jnp.float32,
            )

        dirs = {
            "r": (send_r, recv_r, send_sems_r, recv_sems_r, right, left, 0),
            "l": (send_l, recv_l, send_sems_l, recv_sems_l, left, right, N_HALF),
        }
        credits = {("r", 0): creditA_r, ("r", 1): creditB_r,
                   ("l", 0): creditA_l, ("l", 1): creditB_l}

        def chunk_id(dirname, s):
            if dirname == "r":
                return sigma_ref[jnp.mod(rho - 2 - s, N_DEV)]
            return sigma_ref[jnp.mod(rho + 2 + s, N_DEV)]

        def make_rdma(dirname, sub, slot):
            sb, rb, ss, rs, peer_out, _, _ = dirs[dirname]
            return pltpu.make_async_remote_copy(
                src_ref=sb.at[sub, slot], dst_ref=rb.at[sub, slot],
                send_sem=ss.at[sub, slot], recv_sem=rs.at[sub, slot],
                device_id=(peer_out,), device_id_type=pl.DeviceIdType.MESH)

        last_send = {}

        for dirname in ("r", "l"):
            sb = dirs[dirname][0]
            base = dirs[dirname][6]
            c0 = (sigma_ref[jnp.mod(rho - 1, N_DEV)] if dirname == "r"
                  else sigma_ref[jnp.mod(rho + 1, N_DEV)])
            for sub in (0, 1):
                sb[sub, 0, :, :] = partial(c0, base + sub * N_SUB)
        for dirname in ("r", "l"):
            for sub in (0, 1):
                rd = make_rdma(dirname, sub, 0)
                rd.start()
                last_send[(dirname, sub, 0)] = rd

        y = {}
        for s in range(N_STEPS):
            sp = s % 2
            nsp = (s + 1) % 2
            last = s == N_STEPS - 1
            p = {}
            for dirname in ("r", "l"):
                base = dirs[dirname][6]
                c = chunk_id(dirname, s)
                for sub in (0, 1):
                    p[(dirname, sub)] = partial(c, base + sub * N_SUB)

            for sub in (0, 1):
                for dirname in ("r", "l"):
                    sb, rb, ss, rs, peer_out, peer_in, base = dirs[dirname]
                    make_rdma(dirname, sub, sp).wait_recv()
                    acc = rb[sub, sp, :, :] + p[(dirname, sub)]
                    if not last:
                        prev = last_send.get((dirname, sub, nsp))
                        if prev is not None:
                            prev.wait_send()
                        sb[sub, nsp, :, :] = acc
                        if s >= 1:
                            pl.semaphore_wait(credits[(dirname, sub)], 1)
                        rd = make_rdma(dirname, sub, nsp)
                        rd.start()
                        last_send[(dirname, sub, nsp)] = rd
                        if s <= N_STEPS - 3:
                            pl.semaphore_signal(
                                credits[(dirname, sub)], inc=1,
                                device_id=(peer_in,),
                                device_id_type=pl.DeviceIdType.MESH)
                    else:
                        y[(dirname, sub)] = jnp.maximum(acc, 0.0)

        for key, rd in last_send.items():
            rd.wait_send()

        out_ref[:, 0:N_SUB] = y[("r", 0)]
        out_ref[:, N_SUB:N_HALF] = y[("r", 1)]
        out_ref[:, N_HALF:N_HALF + N_SUB] = y[("l", 0)]
        out_ref[:, N_HALF + N_SUB:] = y[("l", 1)]

        amax = jnp.maximum(
            jnp.maximum(jnp.max(y[("r", 0)]), jnp.max(y[("r", 1)])),
            jnp.maximum(jnp.max(y[("l", 0)]), jnp.max(y[("l", 1)])))
        for r in range(LOG2_DEV):
            partner = jnp.bitwise_xor(d, 1 << r)
            amax_send[r, :, :] = jnp.full((8, 128), amax, jnp.float32)
            ex = pltpu.make_async_remote_copy(
                src_ref=amax_send.at[r], dst_ref=amax_recv.at[r],
                send_sem=amax_send_sems.at[r], recv_sem=amax_recv_sems.at[r],
                device_id=(partner,), device_id_type=pl.DeviceIdType.MESH)
            ex.start()
            ex.wait()
            amax = jnp.maximum(amax, jnp.max(amax_recv[r, :, :]))

        scale = amax / 127.0
        q = jnp.clip(jnp.round(out_ref[:, :] / scale), 0.0, 127.0)
        out_ref[:, :] = q * scale

    sigma_arr = jnp.array(_SIGMA, dtype=jnp.int32)
    inv_arr = jnp.array(_INV, dtype=jnp.int32)

    return pl.pallas_call(
        body,
        out_shape=jax.ShapeDtypeStruct((M_PER, N_COLS), jnp.float32),
        in_specs=[
            pl.BlockSpec(memory_space=pltpu.SMEM),
            pl.BlockSpec(memory_space=pltpu.SMEM),
            pl.BlockSpec(memory_space=pltpu.VMEM),
            pl.BlockSpec(memory_space=pltpu.VMEM),
        ],
        out_specs=pl.BlockSpec(memory_space=pltpu.VMEM),
        scratch_shapes=[
            pltpu.VMEM((2, 2, M_PER, N_SUB), jnp.float32),
            pltpu.VMEM((2, 2, M_PER, N_SUB), jnp.float32),
            pltpu.VMEM((2, 2, M_PER, N_SUB), jnp.float32),
            pltpu.VMEM((2, 2, M_PER, N_SUB), jnp.float32),
            pltpu.SemaphoreType.DMA((2, 2)),
            pltpu.SemaphoreType.DMA((2, 2)),
            pltpu.SemaphoreType.DMA((2, 2)),
            pltpu.SemaphoreType.DMA((2, 2)),
            pltpu.SemaphoreType.REGULAR,
            pltpu.SemaphoreType.REGULAR,
            pltpu.SemaphoreType.REGULAR,
            pltpu.SemaphoreType.REGULAR,
            pltpu.VMEM((LOG2_DEV, 8, 128), jnp.float32),
            pltpu.VMEM((LOG2_DEV, 8, 128), jnp.float32),
            pltpu.SemaphoreType.DMA((LOG2_DEV,)),
            pltpu.SemaphoreType.DMA((LOG2_DEV,)),
        ],
        compiler_params=pltpu.CompilerParams(collective_id=0),
    )(sigma_arr, inv_arr, x, w_mat)
